# baseline (speedup 1.0000x reference)
import functools

import jax
import jax.numpy as jnp
from jax import lax
from jax.experimental import pallas as pl
from jax.experimental.pallas import tpu as pltpu
from jax.experimental.pallas import tpu_sc as plsc

VOCAB = 100000
SIZE = 64
BATCH = 16384
W2 = 10
NEG = 5
NSENSE = NEG + 1
NCOL = 22
NOUT = NSENSE + NEG

NC = 2
NS = 16
NW = NC * NS
BPW = BATCH // NW
CHUNK = 64
NCHUNK = BPW // CHUNK
LANES = 16
QV = SIZE // LANES

CTX_COLS = tuple(range(W2))
SENSE_COLS = (11, 12, 13, 14, 15, 16)
MASK_COLS = (17, 18, 19, 20, 21)
IDX_COLS = CTX_COLS + SENSE_COLS + MASK_COLS
NIDX = len(IDX_COLS)
OFF_BASE = NIDX


def _sc_body(data_hbm, gtab_hbm, stab_hbm, cw_hbm, out_hbm,
             databuf, idxbuf, gbuf, sbuf, cwbuf, outbuf, sem):
    wid = lax.axis_index("s") * NC + lax.axis_index("c")
    base = wid * BPW

    pltpu.sync_copy(data_hbm.at[pl.ds(base, BPW)], databuf)
    pltpu.sync_copy(cw_hbm, cwbuf)

    lane = lax.broadcasted_iota(jnp.int32, (LANES,), 0)

    for g in range(BPW // LANES):
        bvec = lane + (g * LANES)
        c, o = (g * LANES) // CHUNK, (g * LANES) % CHUNK
        for k, col in enumerate(IDX_COLS):
            vals = plsc.load_gather(
                databuf, [bvec, jnp.full((LANES,), col, jnp.int32)])
            if col in SENSE_COLS:
                idxbuf[k, c, pl.ds(o, LANES)] = vals >> 1
                idxbuf[OFF_BASE + (k - W2), c, pl.ds(o, LANES)] = (
                    (vals & 1) << 6)
            else:
                idxbuf[k, c, pl.ds(o, LANES)] = vals

    cwv = [[cwbuf[w, pl.ds(q * LANES, LANES)] for q in range(QV)]
           for w in range(W2)]

    for c in range(NCHUNK):
        copies = []
        for k in range(W2):
            copies.append(pltpu.async_copy(
                gtab_hbm.at[idxbuf.at[k, c]], gbuf.at[k], sem))
        for k in range(NSENSE):
            copies.append(pltpu.async_copy(
                stab_hbm.at[idxbuf.at[W2 + k, c]], sbuf.at[k], sem))
        for cp in copies:
            cp.wait()

        for mi in range(NEG):
            for q in range(CHUNK // LANES):
                mv = idxbuf[W2 + NSENSE + mi, c, pl.ds(q * LANES, LANES)]
                outbuf[NSENSE + mi, pl.ds(q * LANES, LANES)] = (
                    mv.astype(jnp.float32))

        def body(b, ipvecs):
            bi = b & (LANES - 1)
            sel = lane == bi
            acc = []
            for q in range(QV):
                a = gbuf[0, b, pl.ds(q * LANES, LANES)] * cwv[0][q]
                for w in range(1, W2):
                    a = a + gbuf[w, b, pl.ds(q * LANES, LANES)] * cwv[w][q]
                acc.append(a)
            gb = pl.multiple_of(b - bi, LANES)
            biv = jnp.full((LANES,), 0, jnp.int32) + bi
            new = []
            for j in range(NSENSE):
                offv = jnp.take_along_axis(
                    idxbuf[OFF_BASE + j, c, pl.ds(gb, LANES)], biv, axis=0,
                    mode=lax.GatherScatterMode.PROMISE_IN_BOUNDS)
                plo = sbuf[j, b, pl.ds(0, LANES)] * acc[0]
                phi = sbuf[j, b, pl.ds(SIZE, LANES)] * acc[0]
                for q in range(1, QV):
                    plo = plo + sbuf[j, b, pl.ds(q * LANES, LANES)] * acc[q]
                    phi = phi + (sbuf[j, b, pl.ds(SIZE + q * LANES, LANES)]
                                 * acc[q])
                p = jnp.where(offv != 0, phi, plo)
                ip = plsc.cumsum(p)[LANES - 1]
                new.append(jnp.where(sel, ip, ipvecs[j]))

            @pl.when(bi == LANES - 1)
            def _store():
                g0 = pl.multiple_of(b - (LANES - 1), LANES)
                for j in range(NSENSE):
                    outbuf[j, pl.ds(g0, LANES)] = new[j]

            return tuple(new)

        lax.fori_loop(0, CHUNK, body,
                      tuple(jnp.zeros((LANES,), jnp.float32)
                            for _ in range(NSENSE)),
                      unroll=False)
        pltpu.sync_copy(outbuf, out_hbm.at[:, pl.ds(base + c * CHUNK, CHUNK)])


_sc_ips = functools.partial(
    pl.kernel,
    out_type=jax.ShapeDtypeStruct((NOUT, BATCH), jnp.float32),
    mesh=plsc.VectorSubcoreMesh(core_axis_name="c", subcore_axis_name="s"),
    compiler_params=pltpu.CompilerParams(
        needs_layout_passes=False, use_tc_tiling_on_sc=False),
    scratch_types=[
        pltpu.VMEM((BPW, NCOL), jnp.int32),
        pltpu.VMEM((NIDX + NSENSE, NCHUNK, CHUNK), jnp.int32),
        pltpu.VMEM((W2, CHUNK, SIZE), jnp.float32),
        pltpu.VMEM((NSENSE, CHUNK, 2 * SIZE), jnp.float32),
        pltpu.VMEM((W2, SIZE), jnp.float32),
        pltpu.VMEM((NOUT, CHUNK), jnp.float32),
        pltpu.SemaphoreType.DMA,
    ],
)(_sc_body)


def _tc_loss_body(y_ref, o_ref):
    y = y_ref[...]
    pos = jnp.clip(y[0:1, :], -10.0, 10.0)
    neg = jnp.clip(y[1:NSENSE, :], -10.0, 10.0)
    m = y[NSENSE:, :]
    pos_loss = jnp.sum(jnp.log1p(jnp.exp(-pos)), keepdims=True)
    neg_loss = jnp.sum(m * jnp.log1p(jnp.exp(neg)), keepdims=True)
    o_ref[...] = pos_loss + neg_loss


def kernel(data, global_embs, sense_embs, ctx_weight):
    sense128 = sense_embs.reshape(VOCAB, 2 * SIZE)
    y = _sc_ips(data, global_embs, sense128, ctx_weight)
    out = pl.pallas_call(
        _tc_loss_body,
        out_shape=jax.ShapeDtypeStruct((1, 1), jnp.float32),
    )(y)
    return out[0, 0]


# speedup vs baseline: 1.2739x; 1.2739x over previous
import functools

import jax
import jax.numpy as jnp
from jax import lax
from jax.experimental import pallas as pl
from jax.experimental.pallas import tpu as pltpu
from jax.experimental.pallas import tpu_sc as plsc

VOCAB = 100000
SIZE = 64
BATCH = 16384
W2 = 10
NEG = 5
NSENSE = NEG + 1
NCOL = 22

NC = 2
NS = 16
NW = NC * NS
BPW = BATCH // NW
CHUNK = 32
NCHUNK = BPW // CHUNK
LANES = 16
QV = SIZE // LANES

CTX_COLS = tuple(range(W2))
SENSE_COLS = (11, 12, 13, 14, 15, 16)


def _sc_body(dataT_hbm, gtab_hbm, stab_hbm, cw_hbm, out_hbm,
             idxbuf, gbuf, sbuf, cwbuf, outbuf, sem0, sem1):
    wid = lax.axis_index("s") * NC + lax.axis_index("c")
    base = wid * BPW
    sems = (sem0, sem1)

    pltpu.sync_copy(dataT_hbm.at[:, wid], idxbuf)
    pltpu.sync_copy(cw_hbm, cwbuf)

    cwv = [[cwbuf[w, pl.ds(q * LANES, LANES)] for q in range(QV)]
           for w in range(W2)]
    lane = lax.broadcasted_iota(jnp.int32, (LANES,), 0)

    def fire(c):
        pa = c & 1
        copies = []
        for k, col in enumerate(CTX_COLS):
            copies.append(pltpu.async_copy(
                gtab_hbm.at[idxbuf.at[col, c]], gbuf.at[pa, k], sems[pa]))
        for k, col in enumerate(SENSE_COLS):
            copies.append(pltpu.async_copy(
                stab_hbm.at[idxbuf.at[col, c]], sbuf.at[pa, k], sems[pa]))
        return copies

    inflight = fire(0)
    for c in range(NCHUNK):
        pa = c & 1
        for cp in inflight:
            cp.wait()
        if c + 1 < NCHUNK:
            inflight = fire(c + 1)

        def body(b, ipvecs):
            bi = b & (LANES - 1)
            sel = lane == bi
            acc = []
            for q in range(QV):
                a = gbuf[pa, 0, b, pl.ds(q * LANES, LANES)] * cwv[0][q]
                for w in range(1, W2):
                    a = a + gbuf[pa, w, b, pl.ds(q * LANES, LANES)] * cwv[w][q]
                acc.append(a)
            new = []
            for j in range(NSENSE):
                p = sbuf[pa, j, b, pl.ds(0, LANES)] * acc[0]
                for q in range(1, QV):
                    p = p + sbuf[pa, j, b, pl.ds(q * LANES, LANES)] * acc[q]
                ip = plsc.cumsum(p)[LANES - 1]
                new.append(jnp.where(sel, ip, ipvecs[j]))

            @pl.when(bi == LANES - 1)
            def _store():
                g0 = pl.multiple_of(b - (LANES - 1), LANES)
                for j in range(NSENSE):
                    outbuf[j, pl.ds(g0, LANES)] = new[j]

            return tuple(new)

        lax.fori_loop(0, CHUNK, body,
                      tuple(jnp.zeros((LANES,), jnp.float32)
                            for _ in range(NSENSE)),
                      unroll=False)
        pltpu.sync_copy(outbuf, out_hbm.at[:, pl.ds(base + c * CHUNK, CHUNK)])


_sc_ips = functools.partial(
    pl.kernel,
    out_type=jax.ShapeDtypeStruct((NSENSE, BATCH), jnp.float32),
    mesh=plsc.VectorSubcoreMesh(core_axis_name="c", subcore_axis_name="s"),
    compiler_params=pltpu.CompilerParams(
        needs_layout_passes=False, use_tc_tiling_on_sc=False),
    scratch_types=[
        pltpu.VMEM((NCOL, NCHUNK, CHUNK), jnp.int32),
        pltpu.VMEM((2, W2, CHUNK, SIZE), jnp.float32),
        pltpu.VMEM((2, NSENSE, CHUNK, SIZE), jnp.float32),
        pltpu.VMEM((W2, SIZE), jnp.float32),
        pltpu.VMEM((NSENSE, CHUNK), jnp.float32),
        pltpu.SemaphoreType.DMA,
        pltpu.SemaphoreType.DMA,
    ],
)(_sc_body)


def _tc_loss_body(y_ref, m_ref, o_ref):
    y = y_ref[...]
    m = m_ref[...]
    pos = jnp.clip(y[0:1, :], -10.0, 10.0)
    neg = jnp.clip(y[1:NSENSE, :], -10.0, 10.0)
    pos_loss = jnp.sum(jnp.log1p(jnp.exp(-pos)), keepdims=True)
    neg_loss = jnp.sum(m * jnp.log1p(jnp.exp(neg)), keepdims=True)
    o_ref[...] = pos_loss + neg_loss


def kernel(data, global_embs, sense_embs, ctx_weight):
    dataT = data.T.reshape(NCOL, NW, NCHUNK, CHUNK)
    maskf = data[:, W2 + 2 + NEG:].astype(jnp.float32).T

    ips = _sc_ips(dataT, global_embs, sense_embs, ctx_weight)

    out = pl.pallas_call(
        _tc_loss_body,
        out_shape=jax.ShapeDtypeStruct((1, 1), jnp.float32),
    )(ips, maskf)
    return out[0, 0]
